# baseline (speedup 1.0000x reference)
import functools

import jax
import jax.numpy as jnp
from jax import lax
from jax.experimental import pallas as pl
from jax.experimental.pallas import tpu as pltpu
from jax.experimental.pallas import tpu_sc as plsc


def _sc_gather(table, idx, n_rows, D):
    info = plsc.get_sparse_core_info()
    NC, NS = info.num_cores, info.num_subcores
    NW = NC * NS
    CH = 128
    n_g = n_rows // (NW * CH)
    G = 8
    n_grp = n_g // G

    idx3 = idx.reshape(NW, n_g * CH)
    mesh = plsc.VectorSubcoreMesh(core_axis_name="c", subcore_axis_name="s")

    @functools.partial(
        pl.kernel,
        mesh=mesh,
        out_type=jax.ShapeDtypeStruct((NW, n_g, CH, D), jnp.float32),
        scratch_types=[
            pltpu.VMEM((n_g * CH,), jnp.int32),
            pltpu.VMEM((G, CH, D), jnp.float32),
            pltpu.SemaphoreType.DMA,
        ],
        compiler_params=pltpu.CompilerParams(use_tc_tiling_on_sc=False),
    )
    def gather_k(table_hbm, idx_hbm, out_hbm, idx_v, rows_v, sem):
        wid = lax.axis_index("s") * NC + lax.axis_index("c")
        pltpu.sync_copy(idx_hbm.at[wid], idx_v)
        for g in range(n_grp):
            cps = [
                pltpu.async_copy(
                    table_hbm.at[idx_v.at[pl.ds((g * G + j) * CH, CH)]],
                    rows_v.at[j],
                    sem,
                )
                for j in range(G)
            ]
            for cp in cps:
                cp.wait()
            pltpu.sync_copy(rows_v, out_hbm.at[wid, pl.ds(g * G, G)])

    return gather_k(table, idx3).reshape(n_rows, D)


def _tc_mlp(emb, W1, b1, W2, b2, B, L, D, H, E, S):
    def mlp_k(emb_ref, W1_ref, b1_ref, W2_ref, b2_ref, out_ref):
        w1 = W1_ref[...]
        m = jnp.dot(emb_ref[0], w1, preferred_element_type=jnp.float32)
        for l in range(1, L):
            z = jnp.dot(emb_ref[l], w1, preferred_element_type=jnp.float32)
            m = jnp.maximum(m, z)
        m = jnp.maximum(m + b1_ref[...], 0.0)
        out_ref[...] = (
            jnp.dot(m, W2_ref[...], preferred_element_type=jnp.float32)
            + b2_ref[...]
        )

    return pl.pallas_call(
        mlp_k,
        grid=(B // S,),
        in_specs=[
            pl.BlockSpec((L, S, D), lambda i: (0, i, 0)),
            pl.BlockSpec((D, H), lambda i: (0, 0)),
            pl.BlockSpec((1, H), lambda i: (0, 0)),
            pl.BlockSpec((H, E), lambda i: (0, 0)),
            pl.BlockSpec((1, E), lambda i: (0, 0)),
        ],
        out_specs=pl.BlockSpec((S, E), lambda i: (i, 0)),
        out_shape=jax.ShapeDtypeStruct((B, E), jnp.float32),
        compiler_params=pltpu.CompilerParams(
            dimension_semantics=("arbitrary",),
        ),
    )(emb, W1, b1.reshape(1, H), W2, b2.reshape(1, E))


def kernel(x, table, W1, b1, W2, b2):
    B, L = x.shape
    V, D = table.shape
    H = W1.shape[1]
    E = W2.shape[1]

    idx = x.astype(jnp.int32).T.reshape(-1)
    emb = _sc_gather(table, idx, B * L, D)
    return _tc_mlp(emb.reshape(L, B, D), W1, b1, W2, b2, B, L, D, H, E, S=256)
